# Initial kernel scaffold; baseline (speedup 1.0000x reference)
import jax
import jax.numpy as jnp
from jax.experimental import pallas as pl


def kernel(x, W0, s1, U2, V2, W3):
    raise NotImplementedError("write your pallas kernel here")


# speedup vs baseline: 24.2556x; 24.2556x over previous
import jax
import jax.numpy as jnp
from jax.experimental import pallas as pl
from jax.experimental.pallas import tpu as pltpu

_B = 131072
_D = 128
_TILE = 2048


def _body(x_ref, w0_ref, s1_ref, u2_ref, v2_ref, w3_ref, o_ref):
    x = x_ref[...]
    y0 = jax.lax.dot_general(x[:, 0:32], w0_ref[...],
                             (((1,), (1,)), ((), ())),
                             preferred_element_type=jnp.float32)
    y1 = x[:, 32:64] * s1_ref[0, :]
    t = jax.lax.dot_general(x[:, 64:96], v2_ref[...],
                            (((1,), (0,)), ((), ())),
                            preferred_element_type=jnp.float32)
    y2 = jax.lax.dot_general(t, u2_ref[...],
                             (((1,), (1,)), ((), ())),
                             preferred_element_type=jnp.float32)
    y3 = jax.lax.dot_general(x[:, 96:128], w3_ref[...],
                             (((1,), (1,)), ((), ())),
                             preferred_element_type=jnp.float32)
    y = jnp.concatenate([y0, y1, y2, y3], axis=1)
    s = jnp.sum(y * y, axis=1, keepdims=True)
    o_ref[...] = y / (jnp.sqrt(s) + 1e-8)


def kernel(x, W0, s1, U2, V2, W3):
    grid = (_B // _TILE,)
    return pl.pallas_call(
        _body,
        grid=grid,
        in_specs=[
            pl.BlockSpec((_TILE, _D), lambda i: (i, 0)),
            pl.BlockSpec((32, 32), lambda i: (0, 0)),
            pl.BlockSpec((1, 32), lambda i: (0, 0)),
            pl.BlockSpec((32, 8), lambda i: (0, 0)),
            pl.BlockSpec((32, 8), lambda i: (0, 0)),
            pl.BlockSpec((32, 32), lambda i: (0, 0)),
        ],
        out_specs=pl.BlockSpec((_TILE, _D), lambda i: (i, 0)),
        out_shape=jax.ShapeDtypeStruct((_B, _D), jnp.float32),
    )(x, W0, s1.reshape(1, 32), U2, V2, W3)


# speedup vs baseline: 24.7724x; 1.0213x over previous
import jax
import jax.numpy as jnp
from jax.experimental import pallas as pl
from jax.experimental.pallas import tpu as pltpu

_B = 131072
_D = 128
_TILE = 8192


def _body(x_ref, w0_ref, s1_ref, u2_ref, v2_ref, w3_ref, o_ref):
    x = x_ref[...]
    y0 = jax.lax.dot_general(x[:, 0:32], w0_ref[...],
                             (((1,), (1,)), ((), ())),
                             preferred_element_type=jnp.float32)
    y1 = x[:, 32:64] * s1_ref[0, :]
    t = jax.lax.dot_general(x[:, 64:96], v2_ref[...],
                            (((1,), (0,)), ((), ())),
                            preferred_element_type=jnp.float32)
    y2 = jax.lax.dot_general(t, u2_ref[...],
                             (((1,), (1,)), ((), ())),
                             preferred_element_type=jnp.float32)
    y3 = jax.lax.dot_general(x[:, 96:128], w3_ref[...],
                             (((1,), (1,)), ((), ())),
                             preferred_element_type=jnp.float32)
    y = jnp.concatenate([y0, y1, y2, y3], axis=1)
    s = jnp.sum(y * y, axis=1, keepdims=True)
    o_ref[...] = y / (jnp.sqrt(s) + 1e-8)


def kernel(x, W0, s1, U2, V2, W3):
    grid = (_B // _TILE,)
    return pl.pallas_call(
        _body,
        grid=grid,
        in_specs=[
            pl.BlockSpec((_TILE, _D), lambda i: (i, 0)),
            pl.BlockSpec((32, 32), lambda i: (0, 0)),
            pl.BlockSpec((1, 32), lambda i: (0, 0)),
            pl.BlockSpec((32, 8), lambda i: (0, 0)),
            pl.BlockSpec((32, 8), lambda i: (0, 0)),
            pl.BlockSpec((32, 32), lambda i: (0, 0)),
        ],
        out_specs=pl.BlockSpec((_TILE, _D), lambda i: (i, 0)),
        out_shape=jax.ShapeDtypeStruct((_B, _D), jnp.float32),
    )(x, W0, s1.reshape(1, 32), U2, V2, W3)


# speedup vs baseline: 59.3041x; 2.3940x over previous
import functools

import jax
import jax.numpy as jnp
from jax import lax
from jax.experimental import pallas as pl
from jax.experimental.pallas import tpu as pltpu
from jax.experimental.pallas import tpu_sc as plsc

_B = 131072
_D = 128


def _tc_body(x_ref, w0_ref, s1_ref, u2_ref, v2_ref, w3_ref, o_ref, m_ref, ones_ref):
    @pl.when(pl.program_id(0) == 0)
    def _init():
        m_ref[...] = jnp.zeros((_D, _D), dtype=jnp.float32)
        m_ref[0:32, 0:32] = w0_ref[...].T
        r = jax.lax.broadcasted_iota(jnp.int32, (32, 32), 0)
        c = jax.lax.broadcasted_iota(jnp.int32, (32, 32), 1)
        m_ref[32:64, 32:64] = jnp.where(r == c, s1_ref[...], 0.0)
        m_ref[64:96, 64:96] = jax.lax.dot_general(
            v2_ref[...], u2_ref[...], (((1,), (1,)), ((), ())),
            preferred_element_type=jnp.float32)
        m_ref[96:128, 96:128] = w3_ref[...].T
        ones_ref[...] = jnp.ones((_D, _D), dtype=jnp.float32)

    x = x_ref[...]
    y = jax.lax.dot_general(x, m_ref[...], (((1,), (0,)), ((), ())),
                            preferred_element_type=jnp.float32)
    s = jax.lax.dot_general(y * y, ones_ref[...], (((1,), (0,)), ((), ())),
                            preferred_element_type=jnp.float32)
    o_ref[...] = y / (jnp.sqrt(s) + 1e-8)


def _tc_forward(x, W0, s1, U2, V2, W3):
    n = x.shape[0]
    tile = next(t for t in (16384, 8192, 4096, 2048, 1024) if n % t == 0)
    grid = (n // tile,)
    return pl.pallas_call(
        _tc_body,
        grid=grid,
        in_specs=[
            pl.BlockSpec((tile, _D), lambda i: (i, 0)),
            pl.BlockSpec((32, 32), lambda i: (0, 0)),
            pl.BlockSpec((1, 32), lambda i: (0, 0)),
            pl.BlockSpec((32, 8), lambda i: (0, 0)),
            pl.BlockSpec((32, 8), lambda i: (0, 0)),
            pl.BlockSpec((32, 32), lambda i: (0, 0)),
        ],
        out_specs=pl.BlockSpec((tile, _D), lambda i: (i, 0)),
        out_shape=jax.ShapeDtypeStruct((n, _D), jnp.float32),
        compiler_params=pltpu.CompilerParams(
            dimension_semantics=("arbitrary",)),
        scratch_shapes=[
            pltpu.VMEM((_D, _D), jnp.float32),
            pltpu.VMEM((_D, _D), jnp.float32),
        ],
    )(x, W0, s1.reshape(1, 32), U2, V2, W3)


_NC = 2
_NS = 16
_NW = _NC * _NS
_CH = 256

_GDN = lax.GatherDimensionNumbers(
    offset_dims=(), collapsed_slice_dims=(0,), start_index_map=(0,))


def _splat(vec16, lane):
    idx = jnp.full((16, 1), lane, dtype=jnp.int32)
    return lax.gather(vec16, idx, _GDN, (1,),
                      mode=lax.GatherScatterMode.PROMISE_IN_BOUNDS)


def _rsqrt_newton(s):
    i = plsc.bitcast(s, jnp.int32)
    i = 0x5F3759DF - lax.shift_right_logical(i, 1)
    r = plsc.bitcast(i, jnp.float32)
    for _ in range(3):
        r = r * (1.5 - 0.5 * s * r * r)
    return r


def _recip_newton(d):
    i = plsc.bitcast(d, jnp.int32)
    i = 0x7EF311C3 - i
    z = plsc.bitcast(i, jnp.float32)
    for _ in range(3):
        z = z * (2.0 - d * z)
    return z


def _sc_make(n_rows):
    rpw = n_rows // _NW
    ch = min(_CH, rpw)
    nchunk = rpw // ch
    ngroup = ch // 16
    mesh = plsc.VectorSubcoreMesh(core_axis_name="c", subcore_axis_name="s")

    @functools.partial(
        pl.kernel, mesh=mesh,
        out_type=jax.ShapeDtypeStruct((n_rows * _D,), jnp.float32),
        compiler_params=pltpu.CompilerParams(
            use_tc_tiling_on_sc=False, needs_layout_passes=False),
        scratch_types=[
            pltpu.VMEM((ch * _D,), jnp.float32),
            pltpu.VMEM((ch * _D,), jnp.float32),
            pltpu.VMEM((16 * _D,), jnp.float32),
            pltpu.VMEM((32, 32), jnp.float32),
            pltpu.VMEM((32, 32), jnp.float32),
            pltpu.VMEM((32, 16), jnp.float32),
            pltpu.VMEM((32, 16), jnp.float32),
            pltpu.VMEM((32,), jnp.float32),
        ],
    )
    def sck(xf_hbm, w0t_hbm, w3t_hbm, u2p_hbm, v2p_hbm, s1_hbm, of_hbm,
            xv, yv, yt, w0v, w3v, u2v, v2v, s1v):
        wid = lax.axis_index("s") * _NC + lax.axis_index("c")
        pltpu.sync_copy(w0t_hbm, w0v)
        pltpu.sync_copy(w3t_hbm, w3v)
        pltpu.sync_copy(u2p_hbm, u2v)
        pltpu.sync_copy(v2p_hbm, v2v)
        pltpu.sync_copy(s1_hbm, s1v)
        lanes = lax.broadcasted_iota(jnp.int32, (16,), 0)
        voff = lanes * _D

        def chunk_body(c, carry):
            base = (wid * rpw + c * ch) * _D
            pltpu.sync_copy(xf_hbm.at[pl.ds(base, ch * _D)], xv)

            def group_body(g, carry2):
                bvec = voff + g * (16 * _D)
                ssq = jnp.zeros((16,), jnp.float32)

                for colbase, wv in ((0, w0v), (96, w3v)):
                    accs = [jnp.zeros((16,), jnp.float32) for _ in range(32)]
                    for k in range(32):
                        xk = plsc.load_gather(xv, [bvec + (colbase + k)])
                        wlo = wv[k, pl.ds(0, 16)]
                        whi = wv[k, pl.ds(16, 16)]
                        for j in range(32):
                            w = _splat(wlo if j < 16 else whi, j % 16)
                            accs[j] = accs[j] + xk * w
                    for j in range(32):
                        ssq = ssq + accs[j] * accs[j]
                        yt[pl.ds((colbase + j) * 16, 16)] = accs[j]

                s1lo = s1v[pl.ds(0, 16)]
                s1hi = s1v[pl.ds(16, 16)]
                for j in range(32):
                    xk = plsc.load_gather(xv, [bvec + (32 + j)])
                    yj = xk * _splat(s1lo if j < 16 else s1hi, j % 16)
                    ssq = ssq + yj * yj
                    yt[pl.ds((32 + j) * 16, 16)] = yj

                ts = [jnp.zeros((16,), jnp.float32) for _ in range(8)]
                for k in range(32):
                    xk = plsc.load_gather(xv, [bvec + (64 + k)])
                    vrow = v2v[k, pl.ds(0, 16)]
                    for m in range(8):
                        ts[m] = ts[m] + xk * _splat(vrow, m)
                for j in range(32):
                    urow = u2v[j, pl.ds(0, 16)]
                    acc = jnp.zeros((16,), jnp.float32)
                    for m in range(8):
                        acc = acc + ts[m] * _splat(urow, m)
                    ssq = ssq + acc * acc
                    yt[pl.ds((64 + j) * 16, 16)] = acc

                r = _rsqrt_newton(ssq)
                den = ssq * r + 1e-8
                inv = _recip_newton(den)
                for j in range(_D):
                    yj = yt[pl.ds(j * 16, 16)]
                    plsc.store_scatter(yv, [bvec + j], yj * inv)
                return carry2

            lax.fori_loop(0, ngroup, group_body, 0)
            pltpu.sync_copy(yv, of_hbm.at[pl.ds(base, ch * _D)])
            return carry

        lax.fori_loop(0, nchunk, chunk_body, 0)

    return sck


def _sc_forward(x, W0, s1, U2, V2, W3):
    n = x.shape[0]
    sck = _sc_make(n)
    xf = x.reshape(-1)
    w0t = W0.T
    w3t = W3.T
    u2p = jnp.pad(U2, ((0, 0), (0, 8)))
    v2p = jnp.pad(V2, ((0, 0), (0, 8)))
    out = sck(xf, w0t, w3t, u2p, v2p, s1)
    return out.reshape(n, _D)


def kernel(x, W0, s1, U2, V2, W3):
    return _tc_forward(x, W0, s1, U2, V2, W3)
